# baseline (speedup 1.0000x reference)
import functools
import math

import jax
import jax.numpy as jnp
from jax import lax
from jax.experimental import pallas as pl
from jax.experimental.pallas import tpu as pltpu
from jax.experimental.pallas import tpu_sc as plsc

VOCAB = 50265
D = 768
LANES = 16
DJ = D // LANES
NC = 2
NS = 16
NW = NC * NS
INV_SQRT_D = 1.0 / math.sqrt(D)

N = 4 * 4096
PER_W = N // NW
C = 64
NCHUNK = PER_W // C


def _body(widx_hbm, tidx_hbm, sidx_hbm, wtab_hbm, ttab_hbm, stab_hbm, out_hbm,
          widx_v, cidx_v, tvec_v, svec_v, tt_v, st_v, comb_v, rows_v,
          sem0, sem1):
    wid = lax.axis_index("s") * NC + lax.axis_index("c")
    base = wid * PER_W
    sems = (sem0, sem1)

    pltpu.sync_copy(ttab_hbm, tt_v)
    pltpu.sync_copy(stab_hbm, st_v)

    def comb_body(j, carry):
        sl = pl.ds(j * LANES, LANES)
        for t in range(3):
            tv = tt_v[t, sl]
            for s in range(3):
                comb_v[t * 3 + s, sl] = (tv + st_v[s, sl]) * INV_SQRT_D
        return carry

    lax.fori_loop(0, DJ, comb_body, 0)

    def prefetch(g, b):
        start = base + g * C
        pltpu.sync_copy(widx_hbm.at[pl.ds(start, C)], widx_v.at[b])
        pltpu.sync_copy(tidx_hbm.at[pl.ds(start, C)], tvec_v)
        pltpu.sync_copy(sidx_hbm.at[pl.ds(start, C)], svec_v)
        for j in range(C // LANES):
            sl = pl.ds(j * LANES, LANES)
            cidx_v[b, sl] = tvec_v[sl] * 3 + svec_v[sl]
        pltpu.async_copy(wtab_hbm.at[widx_v.at[b]], rows_v.at[b], sems[b])

    def finish(g, b):
        pltpu.make_async_copy(wtab_hbm.at[widx_v.at[b]], rows_v.at[b],
                              sems[b]).wait()

        def grp_body(i16, carry):
            cvec = cidx_v[b, pl.ds(i16 * LANES, LANES)]
            for k in range(LANES):
                cix = cvec[k]
                row = i16 * LANES + k

                def col_body(j, cc):
                    sl = pl.ds(j * LANES, LANES)
                    rows_v[b, row, sl] = rows_v[b, row, sl] + comb_v[cix, sl]
                    return cc

                lax.fori_loop(0, DJ, col_body, 0, unroll=12)
            return carry

        lax.fori_loop(0, C // LANES, grp_body, 0)
        pltpu.sync_copy(rows_v.at[b], out_hbm.at[pl.ds(base + g * C, C)])

    prefetch(0, 0)
    prefetch(1, 1)

    def outer(k, carry):
        g0 = 2 * k
        for b in range(2):
            finish(g0 + b, b)

            @pl.when(k < NCHUNK // 2 - 1)
            def _():
                prefetch(g0 + b + 2, b)
        return carry

    lax.fori_loop(0, NCHUNK // 2, outer, 0)


@jax.jit
def _run(widx, tidx, sidx, wtab, ttab, stab):
    mesh = plsc.VectorSubcoreMesh(core_axis_name="c", subcore_axis_name="s")
    return pl.kernel(
        _body,
        out_type=jax.ShapeDtypeStruct((N, D), jnp.float32),
        mesh=mesh,
        scratch_types=[
            pltpu.VMEM((2, C), jnp.int32),
            pltpu.VMEM((2, C), jnp.int32),
            pltpu.VMEM((C,), jnp.int32),
            pltpu.VMEM((C,), jnp.int32),
            pltpu.VMEM((3, D), jnp.float32),
            pltpu.VMEM((3, D), jnp.float32),
            pltpu.VMEM((9, D), jnp.float32),
            pltpu.VMEM((2, C, D), jnp.float32),
            pltpu.SemaphoreType.DMA,
            pltpu.SemaphoreType.DMA,
        ],
    )(widx, tidx, sidx, wtab, ttab, stab)


def kernel(word_input, position_input, task_input, segment_input,
           word_table, task_table, segment_table):
    del position_input
    B, S = word_input.shape
    widx = word_input.reshape(-1).astype(jnp.int32)
    tidx = task_input.reshape(-1).astype(jnp.int32)
    sidx = segment_input.reshape(-1).astype(jnp.int32)
    out = _run(widx, tidx, sidx, word_table, task_table, segment_table)
    return out.reshape(B, S, D)


# speedup vs baseline: 2.5714x; 2.5714x over previous
import functools
import math

import jax
import jax.numpy as jnp
from jax import lax
from jax.experimental import pallas as pl
from jax.experimental.pallas import tpu as pltpu
from jax.experimental.pallas import tpu_sc as plsc

VOCAB = 50265
D = 768
LANES = 16
DJ = D // LANES
NC = 2
NS = 16
NW = NC * NS
INV_SQRT_D = 1.0 / math.sqrt(D)

N = 4 * 4096
PER_W = N // NW
C = 64
NCHUNK = PER_W // C


def _body(widx_hbm, tidx_hbm, sidx_hbm, wtab_hbm, ttab_hbm, stab_hbm, out_hbm,
          widx_v, cidx_v, tvec_v, svec_v, tt_v, st_v, comb_v, rows_v,
          sem0, sem1):
    wid = lax.axis_index("s") * NC + lax.axis_index("c")
    base = wid * PER_W
    sems = (sem0, sem1)

    pltpu.sync_copy(ttab_hbm, tt_v)
    pltpu.sync_copy(stab_hbm, st_v)

    def comb_body(j, carry):
        sl = pl.ds(j * LANES, LANES)
        for t in range(3):
            tv = tt_v[t, sl]
            for s in range(3):
                comb_v[t * 3 + s, sl] = (tv + st_v[s, sl]) * INV_SQRT_D
        return carry

    lax.fori_loop(0, DJ, comb_body, 0)

    def prefetch(g, b):
        start = base + g * C
        pltpu.sync_copy(widx_hbm.at[pl.ds(start, C)], widx_v.at[b])
        pltpu.sync_copy(tidx_hbm.at[pl.ds(start, C)], tvec_v)
        pltpu.sync_copy(sidx_hbm.at[pl.ds(start, C)], svec_v)
        for j in range(C // LANES):
            sl = pl.ds(j * LANES, LANES)
            cidx_v[b, sl] = tvec_v[sl] * 3 + svec_v[sl]
        pltpu.async_copy(wtab_hbm.at[widx_v.at[b]], rows_v.at[b], sems[b])

    def finish(g, b):
        pltpu.make_async_copy(wtab_hbm.at[widx_v.at[b]], rows_v.at[b],
                              sems[b]).wait()

        def grp_body(i16, carry):
            cvec = cidx_v[b, pl.ds(i16 * LANES, LANES)]
            for k in range(LANES):
                cix = cvec[k]
                row = i16 * LANES + k

                @plsc.parallel_loop(0, DJ, unroll=8)
                def _(j):
                    sl = pl.ds(j * LANES, LANES)
                    rows_v[b, row, sl] = rows_v[b, row, sl] + comb_v[cix, sl]
            return carry

        lax.fori_loop(0, C // LANES, grp_body, 0)
        pltpu.sync_copy(rows_v.at[b], out_hbm.at[pl.ds(base + g * C, C)])

    prefetch(0, 0)
    prefetch(1, 1)

    def outer(k, carry):
        g0 = 2 * k
        for b in range(2):
            finish(g0 + b, b)

            @pl.when(k < NCHUNK // 2 - 1)
            def _():
                prefetch(g0 + b + 2, b)
        return carry

    lax.fori_loop(0, NCHUNK // 2, outer, 0)


@jax.jit
def _run(widx, tidx, sidx, wtab, ttab, stab):
    mesh = plsc.VectorSubcoreMesh(core_axis_name="c", subcore_axis_name="s")
    return pl.kernel(
        _body,
        out_type=jax.ShapeDtypeStruct((N, D), jnp.float32),
        mesh=mesh,
        scratch_types=[
            pltpu.VMEM((2, C), jnp.int32),
            pltpu.VMEM((2, C), jnp.int32),
            pltpu.VMEM((C,), jnp.int32),
            pltpu.VMEM((C,), jnp.int32),
            pltpu.VMEM((3, D), jnp.float32),
            pltpu.VMEM((3, D), jnp.float32),
            pltpu.VMEM((9, D), jnp.float32),
            pltpu.VMEM((2, C, D), jnp.float32),
            pltpu.SemaphoreType.DMA,
            pltpu.SemaphoreType.DMA,
        ],
    )(widx, tidx, sidx, wtab, ttab, stab)


def kernel(word_input, position_input, task_input, segment_input,
           word_table, task_table, segment_table):
    del position_input
    B, S = word_input.shape
    widx = word_input.reshape(-1).astype(jnp.int32)
    tidx = task_input.reshape(-1).astype(jnp.int32)
    sidx = segment_input.reshape(-1).astype(jnp.int32)
    out = _run(widx, tidx, sidx, word_table, task_table, segment_table)
    return out.reshape(B, S, D)


# speedup vs baseline: 2.6189x; 1.0185x over previous
import functools
import math

import jax
import jax.numpy as jnp
from jax import lax
from jax.experimental import pallas as pl
from jax.experimental.pallas import tpu as pltpu
from jax.experimental.pallas import tpu_sc as plsc

VOCAB = 50265
D = 768
LANES = 16
DJ = D // LANES
NC = 2
NS = 16
NW = NC * NS
INV_SQRT_D = 1.0 / math.sqrt(D)

N = 4 * 4096
PER_W = N // NW
C = 64
NCHUNK = PER_W // C


def _body(widx_hbm, tidx_hbm, sidx_hbm, wtab_hbm, ttab_hbm, stab_hbm, out_hbm,
          widx_v, cidx_v, tvec_v, svec_v, tt_v, st_v, comb_v, rows_v,
          sem0, sem1, osem0, osem1):
    wid = lax.axis_index("s") * NC + lax.axis_index("c")
    base = wid * PER_W
    sems = (sem0, sem1)
    osems = (osem0, osem1)

    pltpu.sync_copy(ttab_hbm, tt_v)
    pltpu.sync_copy(stab_hbm, st_v)

    def comb_body(j, carry):
        sl = pl.ds(j * LANES, LANES)
        for t in range(3):
            tv = tt_v[t, sl]
            for s in range(3):
                comb_v[t * 3 + s, sl] = (tv + st_v[s, sl]) * INV_SQRT_D
        return carry

    lax.fori_loop(0, DJ, comb_body, 0)

    def prefetch(g, b, drain=False):
        if drain:
            pltpu.make_async_copy(
                rows_v.at[b], out_hbm.at[pl.ds(base + (g - 2) * C, C)],
                osems[b]).wait()
        start = base + g * C
        pltpu.sync_copy(widx_hbm.at[pl.ds(start, C)], widx_v.at[b])
        pltpu.sync_copy(tidx_hbm.at[pl.ds(start, C)], tvec_v)
        pltpu.sync_copy(sidx_hbm.at[pl.ds(start, C)], svec_v)
        for j in range(C // LANES):
            sl = pl.ds(j * LANES, LANES)
            cidx_v[b, sl] = tvec_v[sl] * 3 + svec_v[sl]
        pltpu.async_copy(wtab_hbm.at[widx_v.at[b]], rows_v.at[b], sems[b])

    def finish(g, b):
        pltpu.make_async_copy(wtab_hbm.at[widx_v.at[b]], rows_v.at[b],
                              sems[b]).wait()

        @plsc.parallel_loop(0, C // LANES)
        def _(i16):
            cvec = cidx_v[b, pl.ds(i16 * LANES, LANES)]
            for k in range(LANES):
                cix = cvec[k]
                row = i16 * LANES + k

                @plsc.parallel_loop(0, DJ, unroll=8)
                def _(j):
                    sl = pl.ds(j * LANES, LANES)
                    rows_v[b, row, sl] = rows_v[b, row, sl] + comb_v[cix, sl]

        pltpu.async_copy(rows_v.at[b], out_hbm.at[pl.ds(base + g * C, C)],
                         osems[b])

    prefetch(0, 0)
    prefetch(1, 1)

    def outer(k, carry):
        g0 = 2 * k
        for b in range(2):
            finish(g0 + b, b)

            @pl.when(k < NCHUNK // 2 - 1)
            def _():
                prefetch(g0 + b + 2, b, drain=True)
        return carry

    lax.fori_loop(0, NCHUNK // 2, outer, 0)

    for b in range(2):
        pltpu.make_async_copy(
            rows_v.at[b], out_hbm.at[pl.ds(base + (NCHUNK - 2 + b) * C, C)],
            osems[b]).wait()


@jax.jit
def _run(widx, tidx, sidx, wtab, ttab, stab):
    mesh = plsc.VectorSubcoreMesh(core_axis_name="c", subcore_axis_name="s")
    return pl.kernel(
        _body,
        out_type=jax.ShapeDtypeStruct((N, D), jnp.float32),
        mesh=mesh,
        scratch_types=[
            pltpu.VMEM((2, C), jnp.int32),
            pltpu.VMEM((2, C), jnp.int32),
            pltpu.VMEM((C,), jnp.int32),
            pltpu.VMEM((C,), jnp.int32),
            pltpu.VMEM((3, D), jnp.float32),
            pltpu.VMEM((3, D), jnp.float32),
            pltpu.VMEM((9, D), jnp.float32),
            pltpu.VMEM((2, C, D), jnp.float32),
            pltpu.SemaphoreType.DMA,
            pltpu.SemaphoreType.DMA,
            pltpu.SemaphoreType.DMA,
            pltpu.SemaphoreType.DMA,
        ],
    )(widx, tidx, sidx, wtab, ttab, stab)


def kernel(word_input, position_input, task_input, segment_input,
           word_table, task_table, segment_table):
    del position_input
    B, S = word_input.shape
    widx = word_input.reshape(-1).astype(jnp.int32)
    tidx = task_input.reshape(-1).astype(jnp.int32)
    sidx = segment_input.reshape(-1).astype(jnp.int32)
    out = _run(widx, tidx, sidx, word_table, task_table, segment_table)
    return out.reshape(B, S, D)
